# speedup vs baseline: 1.2883x; 1.2883x over previous
import functools

import jax
import jax.numpy as jnp
from jax import lax
from jax.experimental import pallas as pl
from jax.experimental.pallas import tpu as pltpu
from jax.experimental.pallas import tpu_sc as plsc

N_ENT = 100000
D_IN = 128
D_OUT = 256
D_REL = 128
B = 16384

ROW_BLK = 2000

NC = 2
NS = 16
NW = NC * NS
BPW = B // NW
CH = 128
NCH = BPW // CH


def _mm_body(a_ref, w_ref, b_ref, o_ref):
    acc = jnp.dot(a_ref[...], w_ref[...], preferred_element_type=jnp.float32)
    o_ref[...] = jnp.tanh(acc + b_ref[...])


def _project(init_embed, pca_weight, pca_bias):
    return pl.pallas_call(
        _mm_body,
        grid=(N_ENT // ROW_BLK,),
        in_specs=[
            pl.BlockSpec((ROW_BLK, D_IN), lambda i: (i, 0)),
            pl.BlockSpec((D_IN, D_OUT), lambda i: (0, 0)),
            pl.BlockSpec((1, D_OUT), lambda i: (0, 0)),
        ],
        out_specs=pl.BlockSpec((ROW_BLK, D_OUT), lambda i: (i, 0)),
        out_shape=jax.ShapeDtypeStruct((N_ENT, D_OUT), jnp.float32),
    )(init_embed, pca_weight, pca_bias.reshape(1, D_OUT))


_sc_mesh = plsc.VectorSubcoreMesh(core_axis_name="c", subcore_axis_name="s")


@functools.partial(
    pl.kernel,
    out_type=(
        jax.ShapeDtypeStruct((B, D_OUT), jnp.float32),
        jax.ShapeDtypeStruct((B, D_REL), jnp.float32),
    ),
    mesh=_sc_mesh,
    scratch_types=[
        pltpu.VMEM((BPW,), jnp.int32),
        pltpu.VMEM((BPW,), jnp.int32),
        pltpu.VMEM((CH, D_OUT), jnp.float32),
        pltpu.VMEM((BPW, D_REL), jnp.float32),
        pltpu.SemaphoreType.DMA,
        pltpu.SemaphoreType.DMA,
    ],
)
def _sc_gathers(x_hbm, sub_hbm, relt_hbm, rel_hbm, sub_out, rel_out,
                sidx, ridx, sbuf, rbuf, ssem, rsem):
    wid = lax.axis_index("s") * NC + lax.axis_index("c")
    base = wid * BPW
    pltpu.sync_copy(sub_hbm.at[pl.ds(base, BPW)], sidx)
    pltpu.sync_copy(rel_hbm.at[pl.ds(base, BPW)], ridx)
    rel_cp = pltpu.async_copy(relt_hbm.at[ridx], rbuf, rsem)
    for c in range(NCH):
        pltpu.async_copy(
            x_hbm.at[sidx.at[pl.ds(c * CH, CH)]], sbuf, ssem
        ).wait()
        pltpu.sync_copy(sbuf, sub_out.at[pl.ds(base + c * CH, CH)])
    rel_cp.wait()
    pltpu.sync_copy(rbuf, rel_out.at[pl.ds(base, BPW)])


def kernel(sub, rel, init_embed, init_rel, pca_weight, pca_bias):
    x2d = _project(init_embed, pca_weight, pca_bias)
    sub_emb, rel_emb = _sc_gathers(x2d, sub.astype(jnp.int32),
                                   init_rel, rel.astype(jnp.int32))
    return sub_emb, rel_emb, x2d.reshape(N_ENT, 2, D_OUT // 2)


# speedup vs baseline: 1.3847x; 1.0748x over previous
import functools

import jax
import jax.numpy as jnp
from jax import lax
from jax.experimental import pallas as pl
from jax.experimental.pallas import tpu as pltpu
from jax.experimental.pallas import tpu_sc as plsc

N_ENT = 100000
D_IN = 128
D_OUT = 256
D_REL = 128
B = 16384

ROW_BLK = 4000

NC = 2
NS = 16
NW = NC * NS
BPW = B // NW
CH = 128
NCH = BPW // CH


def _mm_body(a_ref, w_ref, b_ref, o_ref):
    acc = jnp.dot(a_ref[...], w_ref[...], preferred_element_type=jnp.float32)
    o_ref[...] = jnp.tanh(acc + b_ref[...])


def _project(init_embed, pca_weight, pca_bias):
    return pl.pallas_call(
        _mm_body,
        grid=(N_ENT // ROW_BLK,),
        in_specs=[
            pl.BlockSpec((ROW_BLK, D_IN), lambda i: (i, 0)),
            pl.BlockSpec((D_IN, D_OUT), lambda i: (0, 0)),
            pl.BlockSpec((1, D_OUT), lambda i: (0, 0)),
        ],
        out_specs=pl.BlockSpec((ROW_BLK, D_OUT), lambda i: (i, 0)),
        out_shape=jax.ShapeDtypeStruct((N_ENT, D_OUT), jnp.float32),
    )(init_embed, pca_weight, pca_bias.reshape(1, D_OUT))


_sc_mesh = plsc.VectorSubcoreMesh(core_axis_name="c", subcore_axis_name="s")


@functools.partial(
    pl.kernel,
    out_type=(
        jax.ShapeDtypeStruct((B, D_OUT), jnp.float32),
        jax.ShapeDtypeStruct((B, D_REL), jnp.float32),
    ),
    mesh=_sc_mesh,
    scratch_types=[
        pltpu.VMEM((BPW,), jnp.int32),
        pltpu.VMEM((BPW,), jnp.int32),
        pltpu.VMEM((CH, D_OUT), jnp.float32),
        pltpu.VMEM((BPW, D_REL), jnp.float32),
        pltpu.SemaphoreType.DMA,
        pltpu.SemaphoreType.DMA,
    ],
)
def _sc_gathers(x_hbm, sub_hbm, relt_hbm, rel_hbm, sub_out, rel_out,
                sidx, ridx, sbuf, rbuf, ssem, rsem):
    wid = lax.axis_index("s") * NC + lax.axis_index("c")
    base = wid * BPW
    pltpu.sync_copy(sub_hbm.at[pl.ds(base, BPW)], sidx)
    pltpu.sync_copy(rel_hbm.at[pl.ds(base, BPW)], ridx)
    rel_cp = pltpu.async_copy(relt_hbm.at[ridx], rbuf, rsem)
    for c in range(NCH):
        pltpu.async_copy(
            x_hbm.at[sidx.at[pl.ds(c * CH, CH)]], sbuf, ssem
        ).wait()
        pltpu.sync_copy(sbuf, sub_out.at[pl.ds(base + c * CH, CH)])
    rel_cp.wait()
    pltpu.sync_copy(rbuf, rel_out.at[pl.ds(base, BPW)])


def kernel(sub, rel, init_embed, init_rel, pca_weight, pca_bias):
    x2d = _project(init_embed, pca_weight, pca_bias)
    sub_emb, rel_emb = _sc_gathers(x2d, sub.astype(jnp.int32),
                                   init_rel, rel.astype(jnp.int32))
    return sub_emb, rel_emb, x2d.reshape(N_ENT, 2, D_OUT // 2)


# speedup vs baseline: 1.4078x; 1.0167x over previous
import functools

import jax
import jax.numpy as jnp
from jax import lax
from jax.experimental import pallas as pl
from jax.experimental.pallas import tpu as pltpu
from jax.experimental.pallas import tpu_sc as plsc

N_ENT = 100000
D_IN = 128
D_OUT = 256
D_REL = 128
B = 16384

ROW_BLK = 10000

NC = 2
NS = 16
NW = NC * NS
BPW = B // NW
CH = 128
NCH = BPW // CH


def _mm_body(a_ref, w_ref, b_ref, o_ref):
    acc = jnp.dot(a_ref[...], w_ref[...], preferred_element_type=jnp.float32)
    o_ref[...] = jnp.tanh(acc + b_ref[...])


def _project(init_embed, pca_weight, pca_bias):
    return pl.pallas_call(
        _mm_body,
        grid=(N_ENT // ROW_BLK,),
        in_specs=[
            pl.BlockSpec((ROW_BLK, D_IN), lambda i: (i, 0)),
            pl.BlockSpec((D_IN, D_OUT), lambda i: (0, 0)),
            pl.BlockSpec((1, D_OUT), lambda i: (0, 0)),
        ],
        out_specs=pl.BlockSpec((ROW_BLK, D_OUT), lambda i: (i, 0)),
        out_shape=jax.ShapeDtypeStruct((N_ENT, D_OUT), jnp.float32),
    )(init_embed, pca_weight, pca_bias.reshape(1, D_OUT))


_sc_mesh = plsc.VectorSubcoreMesh(core_axis_name="c", subcore_axis_name="s")


@functools.partial(
    pl.kernel,
    out_type=(
        jax.ShapeDtypeStruct((B, D_OUT), jnp.float32),
        jax.ShapeDtypeStruct((B, D_REL), jnp.float32),
    ),
    mesh=_sc_mesh,
    scratch_types=[
        pltpu.VMEM((BPW,), jnp.int32),
        pltpu.VMEM((BPW,), jnp.int32),
        pltpu.VMEM((CH, D_OUT), jnp.float32),
        pltpu.VMEM((BPW, D_REL), jnp.float32),
        pltpu.SemaphoreType.DMA,
        pltpu.SemaphoreType.DMA,
    ],
)
def _sc_gathers(x_hbm, sub_hbm, relt_hbm, rel_hbm, sub_out, rel_out,
                sidx, ridx, sbuf, rbuf, ssem, rsem):
    wid = lax.axis_index("s") * NC + lax.axis_index("c")
    base = wid * BPW
    pltpu.sync_copy(sub_hbm.at[pl.ds(base, BPW)], sidx)
    pltpu.sync_copy(rel_hbm.at[pl.ds(base, BPW)], ridx)
    rel_cp = pltpu.async_copy(relt_hbm.at[ridx], rbuf, rsem)
    for c in range(NCH):
        pltpu.async_copy(
            x_hbm.at[sidx.at[pl.ds(c * CH, CH)]], sbuf, ssem
        ).wait()
        pltpu.sync_copy(sbuf, sub_out.at[pl.ds(base + c * CH, CH)])
    rel_cp.wait()
    pltpu.sync_copy(rbuf, rel_out.at[pl.ds(base, BPW)])


def kernel(sub, rel, init_embed, init_rel, pca_weight, pca_bias):
    x2d = _project(init_embed, pca_weight, pca_bias)
    sub_emb, rel_emb = _sc_gathers(x2d, sub.astype(jnp.int32),
                                   init_rel, rel.astype(jnp.int32))
    return sub_emb, rel_emb, x2d.reshape(N_ENT, 2, D_OUT // 2)
